# Initial kernel scaffold; baseline (speedup 1.0000x reference)
import jax
import jax.numpy as jnp
from jax.experimental import pallas as pl


def kernel(x, table, W, b):
    raise NotImplementedError("write your pallas kernel here")


# speedup vs baseline: 74.8030x; 74.8030x over previous
import functools

import jax
import jax.numpy as jnp
from jax import lax
from jax.experimental import pallas as pl
from jax.experimental.pallas import tpu as pltpu
from jax.experimental.pallas import tpu_sc as plsc

_VOCAB = 100000
_DIM = 16
_BATCH = 16384
_HIST = 200

_NC = 2
_NS = 16
_NW = _NC * _NS
_B_PER_W = _BATCH // _NW
_CHUNK = 16
_N_CHUNKS = _B_PER_W // _CHUNK
_IDX_PER_CHUNK = _CHUNK * _HIST


def _proj_body(table_ref, w_ref, b_ref, out_ref):
    w128 = jnp.tile(w_ref[...], (1, 8))
    prod = table_ref[...] * w128
    c = lax.broadcasted_iota(jnp.int32, (128, 8), 0)
    j = lax.broadcasted_iota(jnp.int32, (128, 8), 1)
    sel = jnp.where(c // 16 == j, 1.0, 0.0)
    out_ref[...] = (
        jnp.dot(prod, sel, preferred_element_type=jnp.float32) + b_ref[...]
    )


def _project(table, W, b):
    return pl.pallas_call(
        _proj_body,
        out_shape=jax.ShapeDtypeStruct((_VOCAB // 8, 8), jnp.float32),
    )(table.reshape(_VOCAB // 8, 128), W, b.reshape(1, 1))


@functools.partial(
    pl.kernel,
    out_type=jax.ShapeDtypeStruct((_BATCH,), jnp.float32),
    mesh=plsc.VectorSubcoreMesh(core_axis_name="c", subcore_axis_name="s"),
    compiler_params=pltpu.CompilerParams(needs_layout_passes=False),
    scratch_types=[
        pltpu.VMEM((_VOCAB,), jnp.float32),
        pltpu.VMEM((_IDX_PER_CHUNK,), jnp.int32),
        pltpu.VMEM((_B_PER_W,), jnp.float32),
    ],
)
def _sc_pool(t_hbm, x_hbm, out_hbm, t_v, x_v, out_v):
    wid = lax.axis_index("s") * _NC + lax.axis_index("c")

    pltpu.sync_copy(t_hbm, t_v)

    lane_offs = lax.iota(jnp.int32, 16) * _HIST

    def chunk_body(c, carry):
        start = wid * (_B_PER_W * _HIST) + c * _IDX_PER_CHUNK
        pltpu.sync_copy(x_hbm.at[pl.ds(start, _IDX_PER_CHUNK)], x_v)

        def inner(l, acc):
            idxs = plsc.load_gather(x_v, [lane_offs + l])
            vals = plsc.load_gather(t_v, [idxs])
            return acc + vals

        acc = lax.fori_loop(0, _HIST, inner, jnp.zeros((16,), jnp.float32))
        z = acc * (1.0 / _HIST)
        out_v[pl.ds(c * _CHUNK, _CHUNK)] = 1.0 / (1.0 + jnp.exp(-z))
        return carry

    lax.fori_loop(0, _N_CHUNKS, chunk_body, 0)
    pltpu.sync_copy(out_v, out_hbm.at[pl.ds(wid * _B_PER_W, _B_PER_W)])


def kernel(x, table, W, b):
    t = _project(table, W, b).reshape(_VOCAB)
    x_flat = x.reshape(_BATCH * _HIST)
    out = _sc_pool(t, x_flat)
    return out.reshape(_BATCH, 1)


# speedup vs baseline: 101.3759x; 1.3552x over previous
import functools

import jax
import jax.numpy as jnp
from jax import lax
from jax.experimental import pallas as pl
from jax.experimental.pallas import tpu as pltpu
from jax.experimental.pallas import tpu_sc as plsc

_VOCAB = 100000
_DIM = 16
_BATCH = 16384
_HIST = 200

_NC = 2
_NS = 16
_NW = _NC * _NS
_B_PER_W = _BATCH // _NW
_CHUNK = 16
_N_CHUNKS = _B_PER_W // _CHUNK
_UNROLL = 8
_T_ROWS = _VOCAB // 8


def _proj_body(table_ref, w_ref, b_ref, out_ref):
    w128 = jnp.tile(w_ref[...], (1, 8))
    prod = table_ref[...] * w128
    c = lax.broadcasted_iota(jnp.int32, (128, 8), 0)
    j = lax.broadcasted_iota(jnp.int32, (128, 8), 1)
    sel = jnp.where(c // 16 == j, 1.0, 0.0)
    out_ref[...] = (
        jnp.dot(prod, sel, preferred_element_type=jnp.float32) + b_ref[...]
    )


def _project(table, W, b):
    return pl.pallas_call(
        _proj_body,
        out_shape=jax.ShapeDtypeStruct((_T_ROWS, 8), jnp.float32),
    )(table.reshape(_T_ROWS, 128), W, b.reshape(1, 1))


@functools.partial(
    pl.kernel,
    out_type=jax.ShapeDtypeStruct((_BATCH,), jnp.float32),
    mesh=plsc.VectorSubcoreMesh(core_axis_name="c", subcore_axis_name="s"),
    compiler_params=pltpu.CompilerParams(
        needs_layout_passes=False, use_tc_tiling_on_sc=False
    ),
    scratch_types=[
        pltpu.VMEM((_T_ROWS, 8), jnp.float32),
        pltpu.VMEM((2, _CHUNK, _HIST), jnp.int32),
        pltpu.VMEM((_B_PER_W,), jnp.float32),
        pltpu.SemaphoreType.DMA,
        pltpu.SemaphoreType.DMA,
        pltpu.SemaphoreType.DMA,
    ],
)
def _sc_pool(t_hbm, x_hbm, out_hbm, t_v, x_v, out_v, sem0, sem1, sem_t):
    wid = lax.axis_index("s") * _NC + lax.axis_index("c")
    row0 = wid * _B_PER_W

    t_dma = pltpu.async_copy(t_hbm, t_v, sem_t)

    sems = (sem0, sem1)

    def start_fetch(c):
        return pltpu.async_copy(
            x_hbm.at[pl.ds(row0 + c * _CHUNK, _CHUNK), :],
            x_v.at[c % 2],
            sems[c % 2],
        )

    lane = lax.iota(jnp.int32, 16)
    zero = jnp.zeros((16,), jnp.float32)
    izero = jnp.zeros((16,), jnp.int32)

    dmas = [start_fetch(0), None]
    t_dma.wait()

    for c in range(_N_CHUNKS):
        if c + 1 < _N_CHUNKS:
            dmas[(c + 1) % 2] = start_fetch(c + 1)
        dmas[c % 2].wait()
        xc = x_v.at[c % 2]

        def inner(i, accs, xc=xc):
            a0, a1, a2, a3 = accs
            l0 = i * _UNROLL
            for u in range(_UNROLL):
                idx = plsc.load_gather(xc, [lane, izero + (l0 + u)])
                val = plsc.load_gather(
                    t_v,
                    [lax.shift_right_logical(idx, 3), lax.bitwise_and(idx, 7)],
                )
                if u % 4 == 0:
                    a0 = a0 + val
                elif u % 4 == 1:
                    a1 = a1 + val
                elif u % 4 == 2:
                    a2 = a2 + val
                else:
                    a3 = a3 + val
            return a0, a1, a2, a3

        a0, a1, a2, a3 = lax.fori_loop(
            0, _HIST // _UNROLL, inner, (zero, zero, zero, zero)
        )
        z = ((a0 + a1) + (a2 + a3)) * (1.0 / _HIST)
        out_v[pl.ds(c * _CHUNK, _CHUNK)] = 1.0 / (1.0 + jnp.exp(-z))

    pltpu.sync_copy(out_v, out_hbm.at[pl.ds(row0, _B_PER_W)])


def kernel(x, table, W, b):
    t = _project(table, W, b)
    out = _sc_pool(t, x)
    return out.reshape(_BATCH, 1)
